# Initial kernel scaffold; baseline (speedup 1.0000x reference)
import jax
import jax.numpy as jnp
from jax.experimental import pallas as pl


def kernel(x, edge_index, batch, y_pred, gin1_W1, gin1_b1, gin1_W2, gin1_b2, gin2_W1, gin2_b1, gin2_W2, gin2_b2, fus_W1, fus_b1, bn_gamma, bn_beta, fus_W2, fus_b2):
    raise NotImplementedError("write your pallas kernel here")


# speedup vs baseline: 7.2630x; 7.2630x over previous
import functools

import jax
import jax.numpy as jnp
from jax import lax
from jax.experimental import pallas as pl
from jax.experimental.pallas import tpu as pltpu
from jax.experimental.pallas import tpu_sc as plsc

N = 10000
E = 320000
D = 128
G = 128
OUT = 10
H = 256

NC = 2
NS = 16
NW = NC * NS
EPT = E // NW
CH = 80
NCHUNK = EPT // CH
ROWS_PT = 624
TAIL = N - NS * ROWS_PT

BLK = 2000
NB = N // BLK


def _agg_body(h_hbm, zeros_hbm, src_hbm, dst_hbm, out_hbm,
              src_v, dst_v, rows_v, agg_sh, sem):
    c = lax.axis_index("c")
    s = lax.axis_index("s")
    w = c * NS + s
    pltpu.sync_copy(zeros_hbm.at[pl.ds(s * ROWS_PT, ROWS_PT)],
                    agg_sh.at[pl.ds(s * ROWS_PT, ROWS_PT)])

    @pl.when(s == 0)
    def _init_tail():
        pltpu.sync_copy(zeros_hbm.at[pl.ds(NS * ROWS_PT, TAIL)],
                        agg_sh.at[pl.ds(NS * ROWS_PT, TAIL)])
    pltpu.sync_copy(src_hbm.at[w], src_v)
    pltpu.sync_copy(dst_hbm.at[w], dst_v)
    plsc.subcore_barrier()

    def body(i, carry):
        pltpu.async_copy(h_hbm.at[src_v.at[i]], rows_v, sem).wait()
        pltpu.sync_copy(rows_v, agg_sh.at[dst_v.at[i]], add=True)
        return carry

    lax.fori_loop(0, NCHUNK, body, 0)
    plsc.subcore_barrier()
    pltpu.sync_copy(agg_sh.at[pl.ds(s * ROWS_PT, ROWS_PT)],
                    out_hbm.at[pl.ds(c * N + s * ROWS_PT, ROWS_PT)])

    @pl.when(s == 0)
    def _dump_tail():
        pltpu.sync_copy(agg_sh.at[pl.ds(NS * ROWS_PT, TAIL)],
                        out_hbm.at[pl.ds(c * N + NS * ROWS_PT, TAIL)])


def _sc_aggregate(h, zeros, src3, dst3):
    mesh = plsc.VectorSubcoreMesh(core_axis_name="c", subcore_axis_name="s",
                                  num_cores=NC, num_subcores=NS)
    return pl.kernel(
        _agg_body,
        out_type=jax.ShapeDtypeStruct((NC * N, D), jnp.float32),
        mesh=mesh,
        scratch_types=[
            pltpu.VMEM((NCHUNK, CH), jnp.int32),
            pltpu.VMEM((NCHUNK, CH), jnp.int32),
            pltpu.VMEM((CH, D), jnp.float32),
            pltpu.VMEM_SHARED((N, D), jnp.float32),
            pltpu.SemaphoreType.DMA,
        ],
    )(h, zeros, src3, dst3)


def _mlp_kernel(x_ref, a0_ref, a1_ref, w1_ref, b1_ref, w2_ref, b2_ref, o_ref,
                *, final_relu):
    z = x_ref[...] + a0_ref[...] + a1_ref[...]
    z = jnp.dot(z, w1_ref[...], preferred_element_type=jnp.float32) + b1_ref[...]
    z = jnp.maximum(z, 0.0)
    z = jnp.dot(z, w2_ref[...], preferred_element_type=jnp.float32) + b2_ref[...]
    if final_relu:
        z = jnp.maximum(z, 0.0)
    o_ref[...] = z


def _tc_mlp(x, a0, a1, w1, b1, w2, b2, final_relu):
    row = lambda i: (i, 0)
    full = lambda i: (0, 0)
    return pl.pallas_call(
        functools.partial(_mlp_kernel, final_relu=final_relu),
        grid=(NB,),
        in_specs=[
            pl.BlockSpec((BLK, D), row),
            pl.BlockSpec((BLK, D), row),
            pl.BlockSpec((BLK, D), row),
            pl.BlockSpec((D, D), full),
            pl.BlockSpec((1, D), full),
            pl.BlockSpec((D, D), full),
            pl.BlockSpec((1, D), full),
        ],
        out_specs=pl.BlockSpec((BLK, D), row),
        out_shape=jax.ShapeDtypeStruct((N, D), jnp.float32),
    )(x, a0, a1, w1, b1.reshape(1, D), w2, b2.reshape(1, D))


def _final_kernel(h_ref, a0_ref, a1_ref, w1_ref, b1_ref, w2_ref, b2_ref,
                  batch_ref, ypred_ref, fw1a_ref, fw1b_ref, fb1_ref,
                  gamma_ref, beta_ref, fw2_ref, fb2_ref, o_ref,
                  sums_acc, counts_acc):
    i = pl.program_id(0)

    @pl.when(i == 0)
    def _init():
        sums_acc[...] = jnp.zeros_like(sums_acc)
        counts_acc[...] = jnp.zeros_like(counts_acc)

    z = h_ref[...] + a0_ref[...] + a1_ref[...]
    z = jnp.dot(z, w1_ref[...], preferred_element_type=jnp.float32) + b1_ref[...]
    z = jnp.maximum(z, 0.0)
    z = jnp.dot(z, w2_ref[...], preferred_element_type=jnp.float32) + b2_ref[...]
    onehot = (batch_ref[...] ==
              lax.broadcasted_iota(jnp.int32, (BLK, G), 1)).astype(jnp.float32)
    contract = (((0,), (0,)), ((), ()))
    sums_acc[...] += lax.dot_general(onehot, z, contract,
                                     preferred_element_type=jnp.float32,
                                     precision=lax.Precision.HIGHEST)
    counts_acc[...] += lax.dot_general(onehot, jnp.ones((BLK, 128), jnp.float32),
                                       contract,
                                       preferred_element_type=jnp.float32,
                                       precision=lax.Precision.HIGHEST)

    @pl.when(i == NB - 1)
    def _finish():
        counts = jnp.maximum(counts_acc[...], 1.0)
        graph_repr = sums_acc[...] / counts
        z = (jnp.dot(graph_repr, fw1a_ref[...], preferred_element_type=jnp.float32)
             + jnp.dot(ypred_ref[...], fw1b_ref[...], preferred_element_type=jnp.float32)
             + fb1_ref[...])
        mu = jnp.mean(z, axis=0, keepdims=True)
        var = jnp.mean((z - mu) ** 2, axis=0, keepdims=True)
        z = (z - mu) / jnp.sqrt(var + 1e-5) * gamma_ref[...] + beta_ref[...]
        z = jnp.maximum(z, 0.0)
        z = jnp.dot(z, fw2_ref[...], preferred_element_type=jnp.float32) + fb2_ref[...]
        o_ref[...] = jax.nn.sigmoid(z)


def _tc_final(h, a0, a1, w1, b1, w2, b2, batch2, y_pred,
              fw1a, fw1b, fb1, gamma, beta, fw2, fb2):
    row = lambda i: (i, 0)
    full = lambda i: (0, 0)
    return pl.pallas_call(
        _final_kernel,
        grid=(NB,),
        in_specs=[
            pl.BlockSpec((BLK, D), row),
            pl.BlockSpec((BLK, D), row),
            pl.BlockSpec((BLK, D), row),
            pl.BlockSpec((D, D), full),
            pl.BlockSpec((1, D), full),
            pl.BlockSpec((D, D), full),
            pl.BlockSpec((1, D), full),
            pl.BlockSpec((BLK, 1), row),
            pl.BlockSpec((G, OUT), full),
            pl.BlockSpec((D, H), full),
            pl.BlockSpec((OUT, H), full),
            pl.BlockSpec((1, H), full),
            pl.BlockSpec((1, H), full),
            pl.BlockSpec((1, H), full),
            pl.BlockSpec((H, D), full),
            pl.BlockSpec((1, D), full),
        ],
        out_specs=pl.BlockSpec((G, D), full),
        out_shape=jax.ShapeDtypeStruct((G, D), jnp.float32),
        scratch_shapes=[
            pltpu.VMEM((G, D), jnp.float32),
            pltpu.VMEM((G, 128), jnp.float32),
        ],
    )(h, a0, a1, w1, b1.reshape(1, D), w2, b2.reshape(1, D),
      batch2, y_pred, fw1a, fw1b, fb1.reshape(1, H),
      gamma.reshape(1, H), beta.reshape(1, H), fw2, fb2.reshape(1, D))


def kernel(x, edge_index, batch, y_pred,
           gin1_W1, gin1_b1, gin1_W2, gin1_b2,
           gin2_W1, gin2_b1, gin2_W2, gin2_b2,
           fus_W1, fus_b1, bn_gamma, bn_beta, fus_W2, fus_b2):
    src3 = edge_index[0].astype(jnp.int32).reshape(NW, NCHUNK, CH)
    dst3 = edge_index[1].astype(jnp.int32).reshape(NW, NCHUNK, CH)
    zeros = jnp.zeros((N, D), jnp.float32)

    agg1 = _sc_aggregate(x, zeros, src3, dst3)
    h = _tc_mlp(x, agg1[:N], agg1[N:], gin1_W1, gin1_b1, gin1_W2, gin1_b2,
                final_relu=True)
    agg2 = _sc_aggregate(h, zeros, src3, dst3)
    out = _tc_final(h, agg2[:N], agg2[N:], gin2_W1, gin2_b1, gin2_W2, gin2_b2,
                    batch.astype(jnp.int32).reshape(N, 1), y_pred,
                    fus_W1[:D], fus_W1[D:], fus_b1, bn_gamma, bn_beta,
                    fus_W2, fus_b2)
    return out


# speedup vs baseline: 8.8443x; 1.2177x over previous
import functools

import jax
import jax.numpy as jnp
from jax import lax
from jax.experimental import pallas as pl
from jax.experimental.pallas import tpu as pltpu
from jax.experimental.pallas import tpu_sc as plsc

N = 10000
E = 320000
D = 128
G = 128
OUT = 10
H = 256

NC = 2
NS = 16
NW = NC * NS
EPT = E // NW
CH = 80
NCHUNK = EPT // CH
ROWS_PT = 624
TAIL = N - NS * ROWS_PT

BLK = 2000
NB = N // BLK


NBUF = 2
PASS0 = 64
PASS1 = NCHUNK - PASS0


def _agg_body(h_hbm, zeros_hbm, srcA_hbm, dstA_hbm, srcB_hbm, dstB_hbm,
              out_hbm, src_v, dst_v, r0, r1, g0, g1, agg_sh, isem):
    rows = [r0, r1]
    gsems = [g0, g1]
    c = lax.axis_index("c")
    s = lax.axis_index("s")
    w = c * NS + s
    ci = pltpu.async_copy(srcA_hbm.at[w], src_v, isem)
    pltpu.sync_copy(zeros_hbm.at[pl.ds(s * ROWS_PT, ROWS_PT)],
                    agg_sh.at[pl.ds(s * ROWS_PT, ROWS_PT)])

    @pl.when(s == 0)
    def _init_tail():
        pltpu.sync_copy(zeros_hbm.at[pl.ds(NS * ROWS_PT, TAIL)],
                        agg_sh.at[pl.ds(NS * ROWS_PT, TAIL)])
    ci.wait()
    pltpu.sync_copy(dstA_hbm.at[w], dst_v)
    plsc.subcore_barrier()

    def run(count):
        def group(g, carry):
            base = g * NBUF
            gs = [pltpu.async_copy(h_hbm.at[src_v.at[base + k]],
                                   rows[k], gsems[k])
                  for k in range(NBUF)]
            for k in range(NBUF):
                gs[k].wait()
                pltpu.sync_copy(rows[k], agg_sh.at[dst_v.at[base + k]],
                                add=True)
            return carry

        groups = count // NBUF
        lax.fori_loop(0, groups, group, 0)
        for k in range(count - groups * NBUF):
            i = groups * NBUF + k
            pltpu.async_copy(h_hbm.at[src_v.at[i]], rows[k], gsems[k]).wait()
            pltpu.sync_copy(rows[k], agg_sh.at[dst_v.at[i]], add=True)

    run(PASS0)
    pltpu.sync_copy(srcB_hbm.at[w], src_v.at[pl.ds(0, PASS1)])
    pltpu.sync_copy(dstB_hbm.at[w], dst_v.at[pl.ds(0, PASS1)])
    run(PASS1)
    plsc.subcore_barrier()
    pltpu.sync_copy(agg_sh.at[pl.ds(s * ROWS_PT, ROWS_PT)],
                    out_hbm.at[pl.ds(c * N + s * ROWS_PT, ROWS_PT)])

    @pl.when(s == 0)
    def _dump_tail():
        pltpu.sync_copy(agg_sh.at[pl.ds(NS * ROWS_PT, TAIL)],
                        out_hbm.at[pl.ds(c * N + NS * ROWS_PT, TAIL)])


def _sc_aggregate(h, zeros, srcA, dstA, srcB, dstB):
    mesh = plsc.VectorSubcoreMesh(core_axis_name="c", subcore_axis_name="s",
                                  num_cores=NC, num_subcores=NS)
    return pl.kernel(
        _agg_body,
        out_type=jax.ShapeDtypeStruct((NC * N, D), jnp.float32),
        mesh=mesh,
        scratch_types=[
            pltpu.VMEM((PASS0, CH), jnp.int32),
            pltpu.VMEM((PASS0, CH), jnp.int32),
            pltpu.VMEM((CH, D), jnp.float32),
            pltpu.VMEM((CH, D), jnp.float32),
            pltpu.SemaphoreType.DMA,
            pltpu.SemaphoreType.DMA,
            pltpu.VMEM_SHARED((N, D), jnp.float32),
            pltpu.SemaphoreType.DMA,
        ],
    )(h, zeros, srcA, dstA, srcB, dstB)


def _mlp_kernel(x_ref, a0_ref, a1_ref, w1_ref, b1_ref, w2_ref, b2_ref, o_ref,
                *, final_relu):
    z = x_ref[...] + a0_ref[...] + a1_ref[...]
    z = jnp.dot(z, w1_ref[...], preferred_element_type=jnp.float32) + b1_ref[...]
    z = jnp.maximum(z, 0.0)
    z = jnp.dot(z, w2_ref[...], preferred_element_type=jnp.float32) + b2_ref[...]
    if final_relu:
        z = jnp.maximum(z, 0.0)
    o_ref[...] = z


def _tc_mlp(x, a0, a1, w1, b1, w2, b2, final_relu):
    row = lambda i: (i, 0)
    full = lambda i: (0, 0)
    return pl.pallas_call(
        functools.partial(_mlp_kernel, final_relu=final_relu),
        grid=(NB,),
        in_specs=[
            pl.BlockSpec((BLK, D), row),
            pl.BlockSpec((BLK, D), row),
            pl.BlockSpec((BLK, D), row),
            pl.BlockSpec((D, D), full),
            pl.BlockSpec((1, D), full),
            pl.BlockSpec((D, D), full),
            pl.BlockSpec((1, D), full),
        ],
        out_specs=pl.BlockSpec((BLK, D), row),
        out_shape=jax.ShapeDtypeStruct((N, D), jnp.float32),
    )(x, a0, a1, w1, b1.reshape(1, D), w2, b2.reshape(1, D))


def _final_kernel(h_ref, a0_ref, a1_ref, w1_ref, b1_ref, w2_ref, b2_ref,
                  batch_ref, ypred_ref, fw1a_ref, fw1b_ref, fb1_ref,
                  gamma_ref, beta_ref, fw2_ref, fb2_ref, o_ref,
                  sums_acc, counts_acc):
    i = pl.program_id(0)

    @pl.when(i == 0)
    def _init():
        sums_acc[...] = jnp.zeros_like(sums_acc)
        counts_acc[...] = jnp.zeros_like(counts_acc)

    z = h_ref[...] + a0_ref[...] + a1_ref[...]
    z = jnp.dot(z, w1_ref[...], preferred_element_type=jnp.float32) + b1_ref[...]
    z = jnp.maximum(z, 0.0)
    z = jnp.dot(z, w2_ref[...], preferred_element_type=jnp.float32) + b2_ref[...]
    onehot = (batch_ref[...] ==
              lax.broadcasted_iota(jnp.int32, (BLK, G), 1)).astype(jnp.float32)
    contract = (((0,), (0,)), ((), ()))
    sums_acc[...] += lax.dot_general(onehot, z, contract,
                                     preferred_element_type=jnp.float32,
                                     precision=lax.Precision.HIGHEST)
    counts_acc[...] += lax.dot_general(onehot, jnp.ones((BLK, 128), jnp.float32),
                                       contract,
                                       preferred_element_type=jnp.float32,
                                       precision=lax.Precision.HIGHEST)

    @pl.when(i == NB - 1)
    def _finish():
        counts = jnp.maximum(counts_acc[...], 1.0)
        graph_repr = sums_acc[...] / counts
        z = (jnp.dot(graph_repr, fw1a_ref[...], preferred_element_type=jnp.float32)
             + jnp.dot(ypred_ref[...], fw1b_ref[...], preferred_element_type=jnp.float32)
             + fb1_ref[...])
        mu = jnp.mean(z, axis=0, keepdims=True)
        var = jnp.mean((z - mu) ** 2, axis=0, keepdims=True)
        z = (z - mu) / jnp.sqrt(var + 1e-5) * gamma_ref[...] + beta_ref[...]
        z = jnp.maximum(z, 0.0)
        z = jnp.dot(z, fw2_ref[...], preferred_element_type=jnp.float32) + fb2_ref[...]
        o_ref[...] = jax.nn.sigmoid(z)


def _tc_final(h, a0, a1, w1, b1, w2, b2, batch2, y_pred,
              fw1a, fw1b, fb1, gamma, beta, fw2, fb2):
    row = lambda i: (i, 0)
    full = lambda i: (0, 0)
    return pl.pallas_call(
        _final_kernel,
        grid=(NB,),
        in_specs=[
            pl.BlockSpec((BLK, D), row),
            pl.BlockSpec((BLK, D), row),
            pl.BlockSpec((BLK, D), row),
            pl.BlockSpec((D, D), full),
            pl.BlockSpec((1, D), full),
            pl.BlockSpec((D, D), full),
            pl.BlockSpec((1, D), full),
            pl.BlockSpec((BLK, 1), row),
            pl.BlockSpec((G, OUT), full),
            pl.BlockSpec((D, H), full),
            pl.BlockSpec((OUT, H), full),
            pl.BlockSpec((1, H), full),
            pl.BlockSpec((1, H), full),
            pl.BlockSpec((1, H), full),
            pl.BlockSpec((H, D), full),
            pl.BlockSpec((1, D), full),
        ],
        out_specs=pl.BlockSpec((G, D), full),
        out_shape=jax.ShapeDtypeStruct((G, D), jnp.float32),
        scratch_shapes=[
            pltpu.VMEM((G, D), jnp.float32),
            pltpu.VMEM((G, 128), jnp.float32),
        ],
    )(h, a0, a1, w1, b1.reshape(1, D), w2, b2.reshape(1, D),
      batch2, y_pred, fw1a, fw1b, fb1.reshape(1, H),
      gamma.reshape(1, H), beta.reshape(1, H), fw2, fb2.reshape(1, D))


def kernel(x, edge_index, batch, y_pred,
           gin1_W1, gin1_b1, gin1_W2, gin1_b2,
           gin2_W1, gin2_b1, gin2_W2, gin2_b2,
           fus_W1, fus_b1, bn_gamma, bn_beta, fus_W2, fus_b2):
    src3 = edge_index[0].astype(jnp.int32).reshape(NW, NCHUNK, CH)
    dst3 = edge_index[1].astype(jnp.int32).reshape(NW, NCHUNK, CH)
    srcA, srcB = src3[:, :PASS0], src3[:, PASS0:]
    dstA, dstB = dst3[:, :PASS0], dst3[:, PASS0:]
    zeros = jnp.zeros((N, D), jnp.float32)

    agg1 = _sc_aggregate(x, zeros, srcA, dstA, srcB, dstB)
    h = _tc_mlp(x, agg1[:N], agg1[N:], gin1_W1, gin1_b1, gin1_W2, gin1_b2,
                final_relu=True)
    agg2 = _sc_aggregate(h, zeros, srcA, dstA, srcB, dstB)
    out = _tc_final(h, agg2[:N], agg2[N:], gin2_W1, gin2_b1, gin2_W2, gin2_b2,
                    batch.astype(jnp.int32).reshape(N, 1), y_pred,
                    fus_W1[:D], fus_W1[D:], fus_b1, bn_gamma, bn_beta,
                    fus_W2, fus_b2)
    return out


# speedup vs baseline: 9.0476x; 1.0230x over previous
import functools

import jax
import jax.numpy as jnp
from jax import lax
from jax.experimental import pallas as pl
from jax.experimental.pallas import tpu as pltpu
from jax.experimental.pallas import tpu_sc as plsc

N = 10000
E = 320000
D = 128
G = 128
OUT = 10
H = 256

NC = 2
NS = 16
NW = NC * NS
EPT = E // NW
CH = 80
NCHUNK = EPT // CH
ROWS_PT = 624
TAIL = N - NS * ROWS_PT

BLK = 2000
NB = N // BLK


NBUF = 2
PASS0 = 64
PASS1 = NCHUNK - PASS0


def _agg_body(h_hbm, zeros_hbm, srcA_hbm, dstA_hbm, srcB_hbm, dstB_hbm,
              out_hbm, src_v, dst_v, r0, r1, g0, g1, s0, s1, agg_sh, isem):
    rows = [r0, r1]
    gsems = [g0, g1]
    ssems = [s0, s1]
    c = lax.axis_index("c")
    s = lax.axis_index("s")
    w = c * NS + s
    ci = pltpu.async_copy(srcA_hbm.at[w], src_v, isem)
    pltpu.sync_copy(zeros_hbm.at[pl.ds(s * ROWS_PT, ROWS_PT)],
                    agg_sh.at[pl.ds(s * ROWS_PT, ROWS_PT)])

    @pl.when(s == 0)
    def _init_tail():
        pltpu.sync_copy(zeros_hbm.at[pl.ds(NS * ROWS_PT, TAIL)],
                        agg_sh.at[pl.ds(NS * ROWS_PT, TAIL)])
    ci.wait()
    pltpu.sync_copy(dstA_hbm.at[w], dst_v)
    plsc.subcore_barrier()

    def run(count):
        def group(g, carry):
            base = g * NBUF
            gs = [pltpu.async_copy(h_hbm.at[src_v.at[base + k]],
                                   rows[k], gsems[k])
                  for k in range(NBUF)]
            ss = []
            for k in range(NBUF):
                gs[k].wait()
                ss.append(pltpu.async_copy(rows[k],
                                           agg_sh.at[dst_v.at[base + k]],
                                           ssems[k], add=True))
            for k in range(NBUF):
                ss[k].wait()
            return carry

        groups = count // NBUF
        lax.fori_loop(0, groups, group, 0)
        for k in range(count - groups * NBUF):
            i = groups * NBUF + k
            pltpu.async_copy(h_hbm.at[src_v.at[i]], rows[k], gsems[k]).wait()
            pltpu.sync_copy(rows[k], agg_sh.at[dst_v.at[i]], add=True)

    run(PASS0)
    pltpu.sync_copy(srcB_hbm.at[w], src_v.at[pl.ds(0, PASS1)])
    pltpu.sync_copy(dstB_hbm.at[w], dst_v.at[pl.ds(0, PASS1)])
    run(PASS1)
    plsc.subcore_barrier()
    pltpu.sync_copy(agg_sh.at[pl.ds(s * ROWS_PT, ROWS_PT)],
                    out_hbm.at[pl.ds(c * N + s * ROWS_PT, ROWS_PT)])

    @pl.when(s == 0)
    def _dump_tail():
        pltpu.sync_copy(agg_sh.at[pl.ds(NS * ROWS_PT, TAIL)],
                        out_hbm.at[pl.ds(c * N + NS * ROWS_PT, TAIL)])


def _sc_aggregate(h, zeros, srcA, dstA, srcB, dstB):
    mesh = plsc.VectorSubcoreMesh(core_axis_name="c", subcore_axis_name="s",
                                  num_cores=NC, num_subcores=NS)
    return pl.kernel(
        _agg_body,
        out_type=jax.ShapeDtypeStruct((NC * N, D), jnp.float32),
        mesh=mesh,
        scratch_types=[
            pltpu.VMEM((PASS0, CH), jnp.int32),
            pltpu.VMEM((PASS0, CH), jnp.int32),
            pltpu.VMEM((CH, D), jnp.float32),
            pltpu.VMEM((CH, D), jnp.float32),
            pltpu.SemaphoreType.DMA,
            pltpu.SemaphoreType.DMA,
            pltpu.SemaphoreType.DMA,
            pltpu.SemaphoreType.DMA,
            pltpu.VMEM_SHARED((N, D), jnp.float32),
            pltpu.SemaphoreType.DMA,
        ],
    )(h, zeros, srcA, dstA, srcB, dstB)


def _mlp_kernel(x_ref, a0_ref, a1_ref, w1_ref, b1_ref, w2_ref, b2_ref, o_ref,
                *, final_relu):
    z = x_ref[...] + a0_ref[...] + a1_ref[...]
    z = jnp.dot(z, w1_ref[...], preferred_element_type=jnp.float32) + b1_ref[...]
    z = jnp.maximum(z, 0.0)
    z = jnp.dot(z, w2_ref[...], preferred_element_type=jnp.float32) + b2_ref[...]
    if final_relu:
        z = jnp.maximum(z, 0.0)
    o_ref[...] = z


def _tc_mlp(x, a0, a1, w1, b1, w2, b2, final_relu):
    row = lambda i: (i, 0)
    full = lambda i: (0, 0)
    return pl.pallas_call(
        functools.partial(_mlp_kernel, final_relu=final_relu),
        grid=(NB,),
        in_specs=[
            pl.BlockSpec((BLK, D), row),
            pl.BlockSpec((BLK, D), row),
            pl.BlockSpec((BLK, D), row),
            pl.BlockSpec((D, D), full),
            pl.BlockSpec((1, D), full),
            pl.BlockSpec((D, D), full),
            pl.BlockSpec((1, D), full),
        ],
        out_specs=pl.BlockSpec((BLK, D), row),
        out_shape=jax.ShapeDtypeStruct((N, D), jnp.float32),
    )(x, a0, a1, w1, b1.reshape(1, D), w2, b2.reshape(1, D))


def _final_kernel(h_ref, a0_ref, a1_ref, w1_ref, b1_ref, w2_ref, b2_ref,
                  batch_ref, ypred_ref, fw1a_ref, fw1b_ref, fb1_ref,
                  gamma_ref, beta_ref, fw2_ref, fb2_ref, o_ref,
                  sums_acc, counts_acc):
    i = pl.program_id(0)

    @pl.when(i == 0)
    def _init():
        sums_acc[...] = jnp.zeros_like(sums_acc)
        counts_acc[...] = jnp.zeros_like(counts_acc)

    z = h_ref[...] + a0_ref[...] + a1_ref[...]
    z = jnp.dot(z, w1_ref[...], preferred_element_type=jnp.float32) + b1_ref[...]
    z = jnp.maximum(z, 0.0)
    z = jnp.dot(z, w2_ref[...], preferred_element_type=jnp.float32) + b2_ref[...]
    onehot = (batch_ref[...] ==
              lax.broadcasted_iota(jnp.int32, (BLK, G), 1)).astype(jnp.float32)
    contract = (((0,), (0,)), ((), ()))
    sums_acc[...] += lax.dot_general(onehot, z, contract,
                                     preferred_element_type=jnp.float32,
                                     precision=lax.Precision.HIGHEST)
    counts_acc[...] += lax.dot_general(onehot, jnp.ones((BLK, 128), jnp.float32),
                                       contract,
                                       preferred_element_type=jnp.float32,
                                       precision=lax.Precision.HIGHEST)

    @pl.when(i == NB - 1)
    def _finish():
        counts = jnp.maximum(counts_acc[...], 1.0)
        graph_repr = sums_acc[...] / counts
        z = (jnp.dot(graph_repr, fw1a_ref[...], preferred_element_type=jnp.float32)
             + jnp.dot(ypred_ref[...], fw1b_ref[...], preferred_element_type=jnp.float32)
             + fb1_ref[...])
        mu = jnp.mean(z, axis=0, keepdims=True)
        var = jnp.mean((z - mu) ** 2, axis=0, keepdims=True)
        z = (z - mu) / jnp.sqrt(var + 1e-5) * gamma_ref[...] + beta_ref[...]
        z = jnp.maximum(z, 0.0)
        z = jnp.dot(z, fw2_ref[...], preferred_element_type=jnp.float32) + fb2_ref[...]
        o_ref[...] = jax.nn.sigmoid(z)


def _tc_final(h, a0, a1, w1, b1, w2, b2, batch2, y_pred,
              fw1a, fw1b, fb1, gamma, beta, fw2, fb2):
    row = lambda i: (i, 0)
    full = lambda i: (0, 0)
    return pl.pallas_call(
        _final_kernel,
        grid=(NB,),
        in_specs=[
            pl.BlockSpec((BLK, D), row),
            pl.BlockSpec((BLK, D), row),
            pl.BlockSpec((BLK, D), row),
            pl.BlockSpec((D, D), full),
            pl.BlockSpec((1, D), full),
            pl.BlockSpec((D, D), full),
            pl.BlockSpec((1, D), full),
            pl.BlockSpec((BLK, 1), row),
            pl.BlockSpec((G, OUT), full),
            pl.BlockSpec((D, H), full),
            pl.BlockSpec((OUT, H), full),
            pl.BlockSpec((1, H), full),
            pl.BlockSpec((1, H), full),
            pl.BlockSpec((1, H), full),
            pl.BlockSpec((H, D), full),
            pl.BlockSpec((1, D), full),
        ],
        out_specs=pl.BlockSpec((G, D), full),
        out_shape=jax.ShapeDtypeStruct((G, D), jnp.float32),
        scratch_shapes=[
            pltpu.VMEM((G, D), jnp.float32),
            pltpu.VMEM((G, 128), jnp.float32),
        ],
    )(h, a0, a1, w1, b1.reshape(1, D), w2, b2.reshape(1, D),
      batch2, y_pred, fw1a, fw1b, fb1.reshape(1, H),
      gamma.reshape(1, H), beta.reshape(1, H), fw2, fb2.reshape(1, D))


def kernel(x, edge_index, batch, y_pred,
           gin1_W1, gin1_b1, gin1_W2, gin1_b2,
           gin2_W1, gin2_b1, gin2_W2, gin2_b2,
           fus_W1, fus_b1, bn_gamma, bn_beta, fus_W2, fus_b2):
    src3 = edge_index[0].astype(jnp.int32).reshape(NW, NCHUNK, CH)
    dst3 = edge_index[1].astype(jnp.int32).reshape(NW, NCHUNK, CH)
    srcA, srcB = src3[:, :PASS0], src3[:, PASS0:]
    dstA, dstB = dst3[:, :PASS0], dst3[:, PASS0:]
    zeros = jnp.zeros((N, D), jnp.float32)

    agg1 = _sc_aggregate(x, zeros, srcA, dstA, srcB, dstB)
    h = _tc_mlp(x, agg1[:N], agg1[N:], gin1_W1, gin1_b1, gin1_W2, gin1_b2,
                final_relu=True)
    agg2 = _sc_aggregate(h, zeros, srcA, dstA, srcB, dstB)
    out = _tc_final(h, agg2[:N], agg2[N:], gin2_W1, gin2_b1, gin2_W2, gin2_b2,
                    batch.astype(jnp.int32).reshape(N, 1), y_pred,
                    fus_W1[:D], fus_W1[D:], fus_b1, bn_gamma, bn_beta,
                    fus_W2, fus_b2)
    return out


# speedup vs baseline: 9.1687x; 1.0134x over previous
import functools

import jax
import jax.numpy as jnp
from jax import lax
from jax.experimental import pallas as pl
from jax.experimental.pallas import tpu as pltpu
from jax.experimental.pallas import tpu_sc as plsc

N = 10000
E = 320000
D = 128
G = 128
OUT = 10
H = 256

NC = 2
NS = 16
NW = NC * NS
EPT = E // NW
CH = 80
NCHUNK = EPT // CH
ROWS_PT = 624
TAIL = N - NS * ROWS_PT

BLK = 2000
NB = N // BLK


NBUF = 2
PASS0 = 64
PASS1 = NCHUNK - PASS0


def _agg_body(h_hbm, zeros_hbm, srcA_hbm, dstA_hbm, srcB_hbm, dstB_hbm,
              out_hbm, src_v, dst_v, r0, r1, g0, g1, s0, s1, agg_sh, isem):
    rows = [r0, r1]
    gsems = [g0, g1]
    ssems = [s0, s1]
    c = lax.axis_index("c")
    s = lax.axis_index("s")
    w = c * NS + s
    ci = pltpu.async_copy(srcA_hbm.at[w], src_v, isem)
    pltpu.sync_copy(zeros_hbm.at[pl.ds(s * ROWS_PT, ROWS_PT)],
                    agg_sh.at[pl.ds(s * ROWS_PT, ROWS_PT)])

    @pl.when(s == 0)
    def _init_tail():
        pltpu.sync_copy(zeros_hbm.at[pl.ds(NS * ROWS_PT, TAIL)],
                        agg_sh.at[pl.ds(NS * ROWS_PT, TAIL)])
    ci.wait()
    pltpu.sync_copy(dstA_hbm.at[w], dst_v)
    plsc.subcore_barrier()

    def run(count):
        pltpu.async_copy(h_hbm.at[src_v.at[0]], rows[0], gsems[0])
        pltpu.async_copy(h_hbm.at[src_v.at[1]], rows[1], gsems[1])

        def pair(g, carry):
            i0 = 2 * g
            i1 = i0 + 1
            pltpu.make_async_copy(h_hbm.at[src_v.at[i0]],
                                  rows[0], gsems[0]).wait()
            sc0 = pltpu.async_copy(rows[0], agg_sh.at[dst_v.at[i0]],
                                   ssems[0], add=True)
            pltpu.make_async_copy(h_hbm.at[src_v.at[i1]],
                                  rows[1], gsems[1]).wait()
            sc1 = pltpu.async_copy(rows[1], agg_sh.at[dst_v.at[i1]],
                                   ssems[1], add=True)
            sc0.wait()

            @pl.when(i0 + 2 < count)
            def _pref0():
                pltpu.async_copy(h_hbm.at[src_v.at[i0 + 2]],
                                 rows[0], gsems[0])
            sc1.wait()

            @pl.when(i1 + 2 < count)
            def _pref1():
                pltpu.async_copy(h_hbm.at[src_v.at[i1 + 2]],
                                 rows[1], gsems[1])
            return carry

        lax.fori_loop(0, count // 2, pair, 0)
        if count % 2:
            i = count - 1
            pltpu.make_async_copy(h_hbm.at[src_v.at[i]],
                                  rows[0], gsems[0]).wait()
            pltpu.sync_copy(rows[0], agg_sh.at[dst_v.at[i]], add=True)

    run(PASS0)
    pltpu.sync_copy(srcB_hbm.at[w], src_v.at[pl.ds(0, PASS1)])
    pltpu.sync_copy(dstB_hbm.at[w], dst_v.at[pl.ds(0, PASS1)])
    run(PASS1)
    plsc.subcore_barrier()
    pltpu.sync_copy(agg_sh.at[pl.ds(s * ROWS_PT, ROWS_PT)],
                    out_hbm.at[pl.ds(c * N + s * ROWS_PT, ROWS_PT)])

    @pl.when(s == 0)
    def _dump_tail():
        pltpu.sync_copy(agg_sh.at[pl.ds(NS * ROWS_PT, TAIL)],
                        out_hbm.at[pl.ds(c * N + NS * ROWS_PT, TAIL)])


def _sc_aggregate(h, zeros, srcA, dstA, srcB, dstB):
    mesh = plsc.VectorSubcoreMesh(core_axis_name="c", subcore_axis_name="s",
                                  num_cores=NC, num_subcores=NS)
    return pl.kernel(
        _agg_body,
        out_type=jax.ShapeDtypeStruct((NC * N, D), jnp.float32),
        mesh=mesh,
        scratch_types=[
            pltpu.VMEM((PASS0, CH), jnp.int32),
            pltpu.VMEM((PASS0, CH), jnp.int32),
            pltpu.VMEM((CH, D), jnp.float32),
            pltpu.VMEM((CH, D), jnp.float32),
            pltpu.SemaphoreType.DMA,
            pltpu.SemaphoreType.DMA,
            pltpu.SemaphoreType.DMA,
            pltpu.SemaphoreType.DMA,
            pltpu.VMEM_SHARED((N, D), jnp.float32),
            pltpu.SemaphoreType.DMA,
        ],
    )(h, zeros, srcA, dstA, srcB, dstB)


def _mlp_kernel(x_ref, a0_ref, a1_ref, w1_ref, b1_ref, w2_ref, b2_ref, o_ref,
                *, final_relu):
    z = x_ref[...] + a0_ref[...] + a1_ref[...]
    z = jnp.dot(z, w1_ref[...], preferred_element_type=jnp.float32) + b1_ref[...]
    z = jnp.maximum(z, 0.0)
    z = jnp.dot(z, w2_ref[...], preferred_element_type=jnp.float32) + b2_ref[...]
    if final_relu:
        z = jnp.maximum(z, 0.0)
    o_ref[...] = z


def _tc_mlp(x, a0, a1, w1, b1, w2, b2, final_relu):
    row = lambda i: (i, 0)
    full = lambda i: (0, 0)
    return pl.pallas_call(
        functools.partial(_mlp_kernel, final_relu=final_relu),
        grid=(NB,),
        in_specs=[
            pl.BlockSpec((BLK, D), row),
            pl.BlockSpec((BLK, D), row),
            pl.BlockSpec((BLK, D), row),
            pl.BlockSpec((D, D), full),
            pl.BlockSpec((1, D), full),
            pl.BlockSpec((D, D), full),
            pl.BlockSpec((1, D), full),
        ],
        out_specs=pl.BlockSpec((BLK, D), row),
        out_shape=jax.ShapeDtypeStruct((N, D), jnp.float32),
    )(x, a0, a1, w1, b1.reshape(1, D), w2, b2.reshape(1, D))


def _final_kernel(h_ref, a0_ref, a1_ref, w1_ref, b1_ref, w2_ref, b2_ref,
                  batch_ref, ypred_ref, fw1a_ref, fw1b_ref, fb1_ref,
                  gamma_ref, beta_ref, fw2_ref, fb2_ref, o_ref,
                  sums_acc, counts_acc):
    i = pl.program_id(0)

    @pl.when(i == 0)
    def _init():
        sums_acc[...] = jnp.zeros_like(sums_acc)
        counts_acc[...] = jnp.zeros_like(counts_acc)

    z = h_ref[...] + a0_ref[...] + a1_ref[...]
    z = jnp.dot(z, w1_ref[...], preferred_element_type=jnp.float32) + b1_ref[...]
    z = jnp.maximum(z, 0.0)
    z = jnp.dot(z, w2_ref[...], preferred_element_type=jnp.float32) + b2_ref[...]
    onehot = (batch_ref[...] ==
              lax.broadcasted_iota(jnp.int32, (BLK, G), 1)).astype(jnp.float32)
    contract = (((0,), (0,)), ((), ()))
    sums_acc[...] += lax.dot_general(onehot, z, contract,
                                     preferred_element_type=jnp.float32,
                                     precision=lax.Precision.HIGHEST)
    counts_acc[...] += lax.dot_general(onehot, jnp.ones((BLK, 128), jnp.float32),
                                       contract,
                                       preferred_element_type=jnp.float32,
                                       precision=lax.Precision.HIGHEST)

    @pl.when(i == NB - 1)
    def _finish():
        counts = jnp.maximum(counts_acc[...], 1.0)
        graph_repr = sums_acc[...] / counts
        z = (jnp.dot(graph_repr, fw1a_ref[...], preferred_element_type=jnp.float32)
             + jnp.dot(ypred_ref[...], fw1b_ref[...], preferred_element_type=jnp.float32)
             + fb1_ref[...])
        mu = jnp.mean(z, axis=0, keepdims=True)
        var = jnp.mean((z - mu) ** 2, axis=0, keepdims=True)
        z = (z - mu) / jnp.sqrt(var + 1e-5) * gamma_ref[...] + beta_ref[...]
        z = jnp.maximum(z, 0.0)
        z = jnp.dot(z, fw2_ref[...], preferred_element_type=jnp.float32) + fb2_ref[...]
        o_ref[...] = jax.nn.sigmoid(z)


def _tc_final(h, a0, a1, w1, b1, w2, b2, batch2, y_pred,
              fw1a, fw1b, fb1, gamma, beta, fw2, fb2):
    row = lambda i: (i, 0)
    full = lambda i: (0, 0)
    return pl.pallas_call(
        _final_kernel,
        grid=(NB,),
        in_specs=[
            pl.BlockSpec((BLK, D), row),
            pl.BlockSpec((BLK, D), row),
            pl.BlockSpec((BLK, D), row),
            pl.BlockSpec((D, D), full),
            pl.BlockSpec((1, D), full),
            pl.BlockSpec((D, D), full),
            pl.BlockSpec((1, D), full),
            pl.BlockSpec((BLK, 1), row),
            pl.BlockSpec((G, OUT), full),
            pl.BlockSpec((D, H), full),
            pl.BlockSpec((OUT, H), full),
            pl.BlockSpec((1, H), full),
            pl.BlockSpec((1, H), full),
            pl.BlockSpec((1, H), full),
            pl.BlockSpec((H, D), full),
            pl.BlockSpec((1, D), full),
        ],
        out_specs=pl.BlockSpec((G, D), full),
        out_shape=jax.ShapeDtypeStruct((G, D), jnp.float32),
        scratch_shapes=[
            pltpu.VMEM((G, D), jnp.float32),
            pltpu.VMEM((G, 128), jnp.float32),
        ],
    )(h, a0, a1, w1, b1.reshape(1, D), w2, b2.reshape(1, D),
      batch2, y_pred, fw1a, fw1b, fb1.reshape(1, H),
      gamma.reshape(1, H), beta.reshape(1, H), fw2, fb2.reshape(1, D))


def kernel(x, edge_index, batch, y_pred,
           gin1_W1, gin1_b1, gin1_W2, gin1_b2,
           gin2_W1, gin2_b1, gin2_W2, gin2_b2,
           fus_W1, fus_b1, bn_gamma, bn_beta, fus_W2, fus_b2):
    src3 = edge_index[0].astype(jnp.int32).reshape(NW, NCHUNK, CH)
    dst3 = edge_index[1].astype(jnp.int32).reshape(NW, NCHUNK, CH)
    srcA, srcB = src3[:, :PASS0], src3[:, PASS0:]
    dstA, dstB = dst3[:, :PASS0], dst3[:, PASS0:]
    zeros = jnp.zeros((N, D), jnp.float32)

    agg1 = _sc_aggregate(x, zeros, srcA, dstA, srcB, dstB)
    h = _tc_mlp(x, agg1[:N], agg1[N:], gin1_W1, gin1_b1, gin1_W2, gin1_b2,
                final_relu=True)
    agg2 = _sc_aggregate(h, zeros, srcA, dstA, srcB, dstB)
    out = _tc_final(h, agg2[:N], agg2[N:], gin2_W1, gin2_b1, gin2_W2, gin2_b2,
                    batch.astype(jnp.int32).reshape(N, 1), y_pred,
                    fus_W1[:D], fus_W1[D:], fus_b1, bn_gamma, bn_beta,
                    fus_W2, fus_b2)
    return out


# speedup vs baseline: 10.0218x; 1.0930x over previous
import functools

import jax
import jax.numpy as jnp
from jax import lax
from jax.experimental import pallas as pl
from jax.experimental.pallas import tpu as pltpu
from jax.experimental.pallas import tpu_sc as plsc

N = 10000
E = 320000
D = 128
G = 128
OUT = 10
H = 256

NC = 2
NS = 16
NW = NC * NS
EPT = E // NW
CH = 128
NCHUNK = 79
EPAD = NCHUNK * CH - EPT
PADROWS = 8
ROWS_PT = 624
TAIL = N - NS * ROWS_PT

BLK = 2000
NB = N // BLK


NBUF = 2
PASS0 = 40
PASS1 = NCHUNK - PASS0


def _agg_body(h_hbm, zeros_hbm, srcA_hbm, dstA_hbm, srcB_hbm, dstB_hbm,
              out_hbm, src_v, dst_v, r0, r1, g0, g1, s0, s1, agg_sh, isem):
    rows = [r0, r1]
    gsems = [g0, g1]
    ssems = [s0, s1]
    c = lax.axis_index("c")
    s = lax.axis_index("s")
    w = c * NS + s
    ci = pltpu.async_copy(srcA_hbm.at[w], src_v, isem)
    pltpu.sync_copy(zeros_hbm.at[pl.ds(s * ROWS_PT, ROWS_PT)],
                    agg_sh.at[pl.ds(s * ROWS_PT, ROWS_PT)])

    @pl.when(s == 0)
    def _init_tail():
        pltpu.sync_copy(zeros_hbm.at[pl.ds(NS * ROWS_PT, TAIL)],
                        agg_sh.at[pl.ds(NS * ROWS_PT, TAIL)])
    ci.wait()
    pltpu.sync_copy(dstA_hbm.at[w], dst_v)
    plsc.subcore_barrier()

    def run(count):
        pltpu.async_copy(h_hbm.at[src_v.at[0]], rows[0], gsems[0])
        pltpu.async_copy(h_hbm.at[src_v.at[1]], rows[1], gsems[1])

        def pair(g, carry):
            i0 = 2 * g
            i1 = i0 + 1
            pltpu.make_async_copy(h_hbm.at[src_v.at[i0]],
                                  rows[0], gsems[0]).wait()
            sc0 = pltpu.async_copy(rows[0], agg_sh.at[dst_v.at[i0]],
                                   ssems[0], add=True)
            pltpu.make_async_copy(h_hbm.at[src_v.at[i1]],
                                  rows[1], gsems[1]).wait()
            sc1 = pltpu.async_copy(rows[1], agg_sh.at[dst_v.at[i1]],
                                   ssems[1], add=True)
            sc0.wait()

            @pl.when(i0 + 2 < count)
            def _pref0():
                pltpu.async_copy(h_hbm.at[src_v.at[i0 + 2]],
                                 rows[0], gsems[0])
            sc1.wait()

            @pl.when(i1 + 2 < count)
            def _pref1():
                pltpu.async_copy(h_hbm.at[src_v.at[i1 + 2]],
                                 rows[1], gsems[1])
            return carry

        lax.fori_loop(0, count // 2, pair, 0)
        if count % 2:
            i = count - 1
            pltpu.make_async_copy(h_hbm.at[src_v.at[i]],
                                  rows[0], gsems[0]).wait()
            pltpu.sync_copy(rows[0], agg_sh.at[dst_v.at[i]], add=True)

    run(PASS0)
    pltpu.sync_copy(srcB_hbm.at[w], src_v.at[pl.ds(0, PASS1)])
    pltpu.sync_copy(dstB_hbm.at[w], dst_v.at[pl.ds(0, PASS1)])
    run(PASS1)
    plsc.subcore_barrier()
    pltpu.sync_copy(agg_sh.at[pl.ds(s * ROWS_PT, ROWS_PT)],
                    out_hbm.at[pl.ds(c * N + s * ROWS_PT, ROWS_PT)])

    @pl.when(s == 0)
    def _dump_tail():
        pltpu.sync_copy(agg_sh.at[pl.ds(NS * ROWS_PT, TAIL)],
                        out_hbm.at[pl.ds(c * N + NS * ROWS_PT, TAIL)])


def _sc_aggregate(h, zeros, srcA, dstA, srcB, dstB):
    mesh = plsc.VectorSubcoreMesh(core_axis_name="c", subcore_axis_name="s",
                                  num_cores=NC, num_subcores=NS)
    return pl.kernel(
        _agg_body,
        out_type=jax.ShapeDtypeStruct((NC * N, D), jnp.float32),
        mesh=mesh,
        scratch_types=[
            pltpu.VMEM((PASS0, CH), jnp.int32),
            pltpu.VMEM((PASS0, CH), jnp.int32),
            pltpu.VMEM((CH, D), jnp.float32),
            pltpu.VMEM((CH, D), jnp.float32),
            pltpu.SemaphoreType.DMA,
            pltpu.SemaphoreType.DMA,
            pltpu.SemaphoreType.DMA,
            pltpu.SemaphoreType.DMA,
            pltpu.VMEM_SHARED((N + PADROWS, D), jnp.float32),
            pltpu.SemaphoreType.DMA,
        ],
    )(h, zeros, srcA, dstA, srcB, dstB)


def _mlp_kernel(x_ref, a0_ref, a1_ref, w1_ref, b1_ref, w2_ref, b2_ref, o_ref,
                *, final_relu):
    z = x_ref[...] + a0_ref[...] + a1_ref[...]
    z = jnp.dot(z, w1_ref[...], preferred_element_type=jnp.float32) + b1_ref[...]
    z = jnp.maximum(z, 0.0)
    z = jnp.dot(z, w2_ref[...], preferred_element_type=jnp.float32) + b2_ref[...]
    if final_relu:
        z = jnp.maximum(z, 0.0)
    o_ref[...] = z


def _tc_mlp(x, agg, w1, b1, w2, b2, final_relu):
    row = lambda i: (i, 0)
    rowhi = lambda i: (N // BLK + i, 0)
    full = lambda i: (0, 0)
    return pl.pallas_call(
        functools.partial(_mlp_kernel, final_relu=final_relu),
        grid=(NB,),
        in_specs=[
            pl.BlockSpec((BLK, D), row),
            pl.BlockSpec((BLK, D), row),
            pl.BlockSpec((BLK, D), rowhi),
            pl.BlockSpec((D, D), full),
            pl.BlockSpec((1, D), full),
            pl.BlockSpec((D, D), full),
            pl.BlockSpec((1, D), full),
        ],
        out_specs=pl.BlockSpec((BLK, D), row),
        out_shape=jax.ShapeDtypeStruct((N, D), jnp.float32),
    )(x, agg, agg, w1, b1.reshape(1, D), w2, b2.reshape(1, D))


def _final_kernel(h_ref, a0_ref, a1_ref, w1_ref, b1_ref, w2_ref, b2_ref,
                  batch_ref, ypred_ref, fw1a_ref, fw1b_ref, fb1_ref,
                  gamma_ref, beta_ref, fw2_ref, fb2_ref, o_ref,
                  sums_acc, counts_acc):
    i = pl.program_id(0)

    @pl.when(i == 0)
    def _init():
        sums_acc[...] = jnp.zeros_like(sums_acc)
        counts_acc[...] = jnp.zeros_like(counts_acc)

    z = h_ref[...] + a0_ref[...] + a1_ref[...]
    z = jnp.dot(z, w1_ref[...], preferred_element_type=jnp.float32) + b1_ref[...]
    z = jnp.maximum(z, 0.0)
    z = jnp.dot(z, w2_ref[...], preferred_element_type=jnp.float32) + b2_ref[...]
    onehot = (batch_ref[...] ==
              lax.broadcasted_iota(jnp.int32, (BLK, G), 1)).astype(jnp.float32)
    contract = (((0,), (0,)), ((), ()))
    sums_acc[...] += lax.dot_general(onehot, z, contract,
                                     preferred_element_type=jnp.float32,
                                     precision=lax.Precision.HIGHEST)
    counts_acc[...] += lax.dot_general(onehot, jnp.ones((BLK, 128), jnp.float32),
                                       contract,
                                       preferred_element_type=jnp.float32,
                                       precision=lax.Precision.HIGHEST)

    @pl.when(i == NB - 1)
    def _finish():
        counts = jnp.maximum(counts_acc[...], 1.0)
        graph_repr = sums_acc[...] / counts
        z = (jnp.dot(graph_repr, fw1a_ref[...], preferred_element_type=jnp.float32)
             + jnp.dot(ypred_ref[...], fw1b_ref[...], preferred_element_type=jnp.float32)
             + fb1_ref[...])
        mu = jnp.mean(z, axis=0, keepdims=True)
        var = jnp.mean((z - mu) ** 2, axis=0, keepdims=True)
        z = (z - mu) / jnp.sqrt(var + 1e-5) * gamma_ref[...] + beta_ref[...]
        z = jnp.maximum(z, 0.0)
        z = jnp.dot(z, fw2_ref[...], preferred_element_type=jnp.float32) + fb2_ref[...]
        o_ref[...] = jax.nn.sigmoid(z)


def _tc_final(h, agg, w1, b1, w2, b2, batch2, y_pred,
              fw1a, fw1b, fb1, gamma, beta, fw2, fb2):
    row = lambda i: (i, 0)
    rowhi = lambda i: (N // BLK + i, 0)
    full = lambda i: (0, 0)
    return pl.pallas_call(
        _final_kernel,
        grid=(NB,),
        in_specs=[
            pl.BlockSpec((BLK, D), row),
            pl.BlockSpec((BLK, D), row),
            pl.BlockSpec((BLK, D), rowhi),
            pl.BlockSpec((D, D), full),
            pl.BlockSpec((1, D), full),
            pl.BlockSpec((D, D), full),
            pl.BlockSpec((1, D), full),
            pl.BlockSpec((BLK, 1), row),
            pl.BlockSpec((G, OUT), full),
            pl.BlockSpec((D, H), full),
            pl.BlockSpec((OUT, H), full),
            pl.BlockSpec((1, H), full),
            pl.BlockSpec((1, H), full),
            pl.BlockSpec((1, H), full),
            pl.BlockSpec((H, D), full),
            pl.BlockSpec((1, D), full),
        ],
        out_specs=pl.BlockSpec((G, D), full),
        out_shape=jax.ShapeDtypeStruct((G, D), jnp.float32),
        scratch_shapes=[
            pltpu.VMEM((G, D), jnp.float32),
            pltpu.VMEM((G, 128), jnp.float32),
        ],
    )(h, agg, agg, w1, b1.reshape(1, D), w2, b2.reshape(1, D),
      batch2, y_pred, fw1a, fw1b, fb1.reshape(1, H),
      gamma.reshape(1, H), beta.reshape(1, H), fw2, fb2.reshape(1, D))


def kernel(x, edge_index, batch, y_pred,
           gin1_W1, gin1_b1, gin1_W2, gin1_b2,
           gin2_W1, gin2_b1, gin2_W2, gin2_b2,
           fus_W1, fus_b1, bn_gamma, bn_beta, fus_W2, fus_b2):
    src2 = edge_index[0].astype(jnp.int32).reshape(NW, EPT)
    dst2 = edge_index[1].astype(jnp.int32).reshape(NW, EPT)
    pad_s = jnp.broadcast_to((jnp.arange(EPAD, dtype=jnp.int32) * 89) % N,
                             (NW, EPAD))
    pad_d = jnp.broadcast_to(N + (jnp.arange(EPAD, dtype=jnp.int32) % PADROWS),
                             (NW, EPAD))
    src3 = jnp.concatenate([src2, pad_s], axis=1).reshape(NW, NCHUNK, CH)
    dst3 = jnp.concatenate([dst2, pad_d], axis=1).reshape(NW, NCHUNK, CH)
    srcA, srcB = src3[:, :PASS0], src3[:, PASS0:]
    dstA, dstB = dst3[:, :PASS0], dst3[:, PASS0:]
    zeros = jnp.zeros((N, D), jnp.float32)

    agg1 = _sc_aggregate(x, zeros, srcA, dstA, srcB, dstB)
    h = _tc_mlp(x, agg1, gin1_W1, gin1_b1, gin1_W2, gin1_b2,
                final_relu=True)
    agg2 = _sc_aggregate(h, zeros, srcA, dstA, srcB, dstB)
    out = _tc_final(h, agg2, gin2_W1, gin2_b1, gin2_W2, gin2_b2,
                    batch.astype(jnp.int32).reshape(N, 1), y_pred,
                    fus_W1[:D], fus_W1[D:], fus_b1, bn_gamma, bn_beta,
                    fus_W2, fus_b2)
    return out
